# baseline (speedup 1.0000x reference)
import functools

import jax
import jax.numpy as jnp
from jax import lax
from jax.experimental import pallas as pl
from jax.experimental.pallas import tpu as pltpu
from jax.experimental.pallas import tpu_sc as plsc

BATCH = 4096
SEQ = 200
D = 32
NC, NS = 2, 16
NW = NC * NS
BPW = BATCH // NW
CB = 8
NCHUNK = BPW // CB
GW = 40
NG = SEQ // GW

_mesh = plsc.VectorSubcoreMesh(core_axis_name="c", subcore_axis_name="s")


@functools.partial(
    pl.kernel,
    mesh=_mesh,
    out_type=jax.ShapeDtypeStruct((BATCH, SEQ, D), jnp.float32),
    scratch_types=[
        pltpu.VMEM((CB, NG, GW), jnp.int32),
        pltpu.VMEM((SEQ, 128), jnp.float32),
        pltpu.VMEM((SEQ, D), jnp.float32),
        pltpu.VMEM((SEQ, D), jnp.float32),
        pltpu.SemaphoreType.DMA,
    ],
)
def _emb_lookup(qx_hbm, item_hbm, pos_hbm, out_hbm, qidx_v, gbuf, obuf,
                pos_v, sem):
    wid = lax.axis_index("s") * NC + lax.axis_index("c")
    pltpu.sync_copy(pos_hbm, pos_v)
    base_b = wid * BPW

    def chunk_body(c, carry):
        b0 = base_b + c * CB
        pltpu.sync_copy(qx_hbm.at[pl.ds(b0, CB)], qidx_v)

        def row_body(b, carry2):
            copies = [
                pltpu.async_copy(item_hbm.at[qidx_v.at[b, h]],
                                 gbuf.at[pl.ds(h * GW, GW)], sem)
                for h in range(NG)
            ]
            for cp in copies:
                cp.wait()

            def extract_body(l, carry3):
                g0 = gbuf[l, pl.ds(0, 16)]
                g1 = gbuf[l, pl.ds(16, 16)]
                obuf[l, pl.ds(0, 16)] = g0 + pos_v[l, pl.ds(0, 16)]
                obuf[l, pl.ds(16, 16)] = g1 + pos_v[l, pl.ds(16, 16)]
                return carry3

            lax.fori_loop(0, SEQ, extract_body, 0)
            pltpu.sync_copy(obuf, out_hbm.at[b0 + b])
            return carry2

        lax.fori_loop(0, CB, row_body, 0)
        return carry

    lax.fori_loop(0, NCHUNK, chunk_body, 0)


def kernel(x, item_emb, pos_emb):
    item128 = jnp.pad(item_emb, ((0, 0), (0, 128 - D)))
    qx = x.reshape(BATCH, NG, GW)
    return _emb_lookup(qx, item128, pos_emb)
